# Initial kernel scaffold; baseline (speedup 1.0000x reference)
import jax
import jax.numpy as jnp
from jax.experimental import pallas as pl


def kernel(x, edge_index, W1, b1, W2, b2):
    raise NotImplementedError("write your pallas kernel here")


# speedup vs baseline: 4.2174x; 4.2174x over previous
import functools

import jax
import jax.numpy as jnp
from jax import lax
from jax.experimental import pallas as pl
from jax.experimental.pallas import tpu as pltpu
from jax.experimental.pallas import tpu_sc as plsc

N = 10000
E = 320000
D = 128

NC = 2
NS = 16
NW = NC * NS

N_PAD = 10240
ROWS_PER_SUB = N_PAD // NS
E_PER_TILE = E // NW
K = 80
CHUNKS = E_PER_TILE // K

_mesh = plsc.VectorSubcoreMesh(core_axis_name="c", subcore_axis_name="s")


E_PER_SUB = E // NS
CHUNKS_DEG = E_PER_SUB // K


def _sc_degrees(src, dst, zeros128, ones128):
    @functools.partial(
        pl.kernel,
        out_type=jax.ShapeDtypeStruct((NC, N_PAD, D), jnp.float32),
        mesh=_mesh,
        scratch_types=[
            pltpu.VMEM((K,), jnp.int32),
            pltpu.VMEM((K, D), jnp.float32),
            pltpu.VMEM_SHARED((N_PAD, D), jnp.float32),
        ],
    )
    def k(src_hbm, dst_hbm, z_hbm, o_hbm, deg_hbm, idx_v, ones_v, acc_sh):
        c = lax.axis_index("c")
        s = lax.axis_index("s")
        row0 = s * ROWS_PER_SUB
        pltpu.sync_copy(z_hbm, acc_sh.at[pl.ds(row0, ROWS_PER_SUB)])
        pltpu.sync_copy(o_hbm, ones_v)
        plsc.subcore_barrier()

        def accumulate(idx_hbm):
            @pl.loop(0, CHUNKS_DEG)
            def _(i):
                base = s * E_PER_SUB + i * K
                pltpu.sync_copy(idx_hbm.at[pl.ds(base, K)], idx_v)
                pltpu.sync_copy(ones_v, acc_sh.at[idx_v], add=True)

        @pl.when(c == 0)
        def _():
            accumulate(src_hbm)

        @pl.when(c == 1)
        def _():
            accumulate(dst_hbm)

        plsc.subcore_barrier()
        pltpu.sync_copy(acc_sh.at[pl.ds(row0, ROWS_PER_SUB)],
                        deg_hbm.at[c, pl.ds(row0, ROWS_PER_SUB)])

    return k(src, dst, zeros128, ones128)


def _sc_msgpass(table, src, dst, zeros128):
    @functools.partial(
        pl.kernel,
        out_type=jax.ShapeDtypeStruct((NC, N_PAD, D), jnp.float32),
        mesh=_mesh,
        scratch_types=[
            pltpu.VMEM((K,), jnp.int32),
            pltpu.VMEM((K,), jnp.int32),
            pltpu.VMEM((K, D), jnp.float32),
            pltpu.VMEM_SHARED((N_PAD, D), jnp.float32),
        ],
    )
    def k(t_hbm, src_hbm, dst_hbm, z_hbm, out_hbm,
          src_v, dst_v, rows_v, acc_sh):
        c = lax.axis_index("c")
        s = lax.axis_index("s")
        wid = s * NC + c
        row0 = s * ROWS_PER_SUB
        pltpu.sync_copy(z_hbm, acc_sh.at[pl.ds(row0, ROWS_PER_SUB)])
        plsc.subcore_barrier()

        @pl.loop(0, CHUNKS)
        def _(i):
            base = wid * E_PER_TILE + i * K
            pltpu.sync_copy(src_hbm.at[pl.ds(base, K)], src_v)
            pltpu.sync_copy(dst_hbm.at[pl.ds(base, K)], dst_v)
            pltpu.sync_copy(t_hbm.at[src_v], rows_v)
            pltpu.sync_copy(rows_v, acc_sh.at[dst_v], add=True)

        plsc.subcore_barrier()
        pltpu.sync_copy(acc_sh.at[pl.ds(row0, ROWS_PER_SUB)],
                        out_hbm.at[c, pl.ds(row0, ROWS_PER_SUB)])

    return k(table, src, dst, zeros128)


BR = 400
NBLK = N // BR


def _norm_from_deg(dref):
    d0 = dref[:, 0:1]
    return jnp.where(d0 > 0.0, lax.rsqrt(jnp.maximum(d0, 1.0)), 0.0)


def _tc_prescale_body(x_ref, dgo_ref, o_ref):
    ns = _norm_from_deg(dgo_ref[...])
    o_ref[...] = x_ref[...] * ns


def _tc_prescale(x, dgo):
    return pl.pallas_call(
        _tc_prescale_body,
        grid=(NBLK,),
        in_specs=[
            pl.BlockSpec((BR, D), lambda i: (i, 0)),
            pl.BlockSpec((BR, D), lambda i: (i, 0)),
        ],
        out_specs=pl.BlockSpec((BR, D), lambda i: (i, 0)),
        out_shape=jax.ShapeDtypeStruct((N, D), jnp.float32),
    )(x, dgo)


def _tc_mid_body(p_ref, dgi_ref, dgo_ref, w_ref, b_ref, o_ref):
    nd = _norm_from_deg(dgi_ref[...])
    agg = (p_ref[0] + p_ref[1]) * nd
    y = jnp.dot(agg, w_ref[...], preferred_element_type=jnp.float32)
    h = jnp.maximum(y + b_ref[...], 0.0)
    ns = _norm_from_deg(dgo_ref[...])
    o_ref[...] = h * ns


def _tc_mid(p, dgi, dgo, w, b):
    return pl.pallas_call(
        _tc_mid_body,
        grid=(NBLK,),
        in_specs=[
            pl.BlockSpec((NC, BR, D), lambda i: (0, i, 0)),
            pl.BlockSpec((BR, D), lambda i: (i, 0)),
            pl.BlockSpec((BR, D), lambda i: (i, 0)),
            pl.BlockSpec((D, D), lambda i: (0, 0)),
            pl.BlockSpec((1, D), lambda i: (0, 0)),
        ],
        out_specs=pl.BlockSpec((BR, D), lambda i: (i, 0)),
        out_shape=jax.ShapeDtypeStruct((N, D), jnp.float32),
    )(p, dgi, dgo, w, b)


def _tc_final_body(p_ref, dgi_ref, w_ref, b_ref, o_ref):
    nd = _norm_from_deg(dgi_ref[...])
    agg = (p_ref[0] + p_ref[1]) * nd
    y = jnp.dot(agg, w_ref[...], preferred_element_type=jnp.float32)
    o_ref[...] = jnp.maximum(y + b_ref[...], 0.0)


def _tc_final(p, dgi, w, b):
    return pl.pallas_call(
        _tc_final_body,
        grid=(NBLK,),
        in_specs=[
            pl.BlockSpec((NC, BR, D), lambda i: (0, i, 0)),
            pl.BlockSpec((BR, D), lambda i: (i, 0)),
            pl.BlockSpec((D, D), lambda i: (0, 0)),
            pl.BlockSpec((1, D), lambda i: (0, 0)),
        ],
        out_specs=pl.BlockSpec((BR, D), lambda i: (i, 0)),
        out_shape=jax.ShapeDtypeStruct((N, D), jnp.float32),
    )(p, dgi, w, b)


def kernel(x, edge_index, W1, b1, W2, b2):
    src = edge_index[0]
    dst = edge_index[1]
    ones128 = jnp.ones((K, D), jnp.float32)
    zeros128 = jnp.zeros((ROWS_PER_SUB, D), jnp.float32)
    b1r = b1.reshape(1, D)
    b2r = b2.reshape(1, D)

    deg = _sc_degrees(src, dst, zeros128, ones128)
    dgo = deg[0]
    dgi = deg[1]

    t0 = _tc_prescale(x, dgo)
    p1 = _sc_msgpass(t0, src, dst, zeros128)
    t1 = _tc_mid(p1, dgi, dgo, W1, b1r)
    p2 = _sc_msgpass(t1, src, dst, zeros128)
    out = _tc_final(p2, dgi, W2, b2r)
    return out
